# Initial kernel scaffold; baseline (speedup 1.0000x reference)
import jax
import jax.numpy as jnp
from jax.experimental import pallas as pl


def kernel(hidden_states, attention_mask, Wq, bq, Wk, bk, Wv, bv, Wo, bo, ln1_g, ln1_b, Wr, br, Wi, bi, Wout, bout, ln2_g, ln2_b):
    raise NotImplementedError("write your pallas kernel here")


# speedup vs baseline: 2.0060x; 2.0060x over previous
import functools

import jax
import jax.numpy as jnp
from jax import lax
from jax.experimental import pallas as pl
from jax.experimental.pallas import tpu as pltpu
from jax.experimental.pallas import tpu_sc as plsc

S, D, H, DH, DFF, E = 2048, 768, 12, 64, 3072, 8
EPS = 1e-12
TB = 256
BLK = 128
NBLK = 23
SP = (NBLK + 1) * BLK
NW = 32
RPW = S // NW

_PREC = jax.lax.Precision.DEFAULT


def _dot(a, b):
    return jax.lax.dot_general(a, b, (((1,), (0,)), ((), ())),
                               preferred_element_type=jnp.float32,
                               precision=_PREC)


def _ln(t, g, b):
    m = jnp.mean(t, axis=-1, keepdims=True)
    v = jnp.mean((t - m) ** 2, axis=-1, keepdims=True)
    return (t - m) / jnp.sqrt(v + EPS) * g + b


def _qkv_body(x_ref, w_ref, b_ref, o_ref):
    o_ref[...] = _dot(x_ref[...], w_ref[...]) + b_ref[...]


def _qkv_call(x, wqkv, bqkv):
    return pl.pallas_call(
        _qkv_body,
        grid=(S // TB,),
        in_specs=[
            pl.BlockSpec((TB, D), lambda i: (i, 0)),
            pl.BlockSpec((D, 3 * D), lambda i: (0, 0)),
            pl.BlockSpec((1, 3 * D), lambda i: (0, 0)),
        ],
        out_specs=pl.BlockSpec((TB, 3 * D), lambda i: (i, 0)),
        out_shape=jax.ShapeDtypeStruct((S, 3 * D), jnp.float32),
    )(x, wqkv, bqkv)


ACH = 1024


def _attn_body(q_ref, k_ref, v_ref, m_ref, o_ref):
    q2 = q_ref[...]
    msk = m_ref[...]
    for t in range(2):
        q = q2[:, t * DH:(t + 1) * DH]
        o = m = sm = None
        for c in range(S // ACH):
            kc = k_ref[pl.ds(c * ACH, ACH), t * DH:(t + 1) * DH]
            vc = v_ref[pl.ds(c * ACH, ACH), t * DH:(t + 1) * DH]
            s = jax.lax.dot_general(q, kc, (((1,), (1,)), ((), ())),
                                    preferred_element_type=jnp.float32,
                                    precision=_PREC)
            s = s * (1.0 / 8.0) + msk[:, c * ACH:(c + 1) * ACH]
            cmx = jnp.max(s, axis=-1, keepdims=True)
            if o is None:
                m = cmx
                e = jnp.exp(s - m)
                sm = jnp.sum(e, axis=-1, keepdims=True)
                o = _dot(e, vc)
            else:
                mn = jnp.maximum(m, cmx)
                corr = jnp.exp(m - mn)
                e = jnp.exp(s - mn)
                sm = sm * corr + jnp.sum(e, axis=-1, keepdims=True)
                o = o * corr + _dot(e, vc)
                m = mn
        o_ref[:, t * DH:(t + 1) * DH] = o / sm


def _attn_call(qkv, mask):
    return pl.pallas_call(
        _attn_body,
        grid=(H // 2, S // TB),
        in_specs=[
            pl.BlockSpec((TB, 2 * DH), lambda h, i: (i, h)),
            pl.BlockSpec((S, 2 * DH), lambda h, i: (0, 6 + h)),
            pl.BlockSpec((S, 2 * DH), lambda h, i: (0, 12 + h)),
            pl.BlockSpec((1, S), lambda h, i: (0, 0)),
        ],
        out_specs=pl.BlockSpec((TB, 2 * DH), lambda h, i: (i, h)),
        out_shape=jax.ShapeDtypeStruct((S, D), jnp.float32),
    )(qkv, qkv, qkv, mask)


def _proj_body(ctx_ref, h_ref, wo_ref, bo_ref, g_ref, b_ref, wr_ref, br_ref,
               ao_ref, rs_ref):
    t = _dot(ctx_ref[...], wo_ref[...]) + bo_ref[...] + h_ref[...]
    ao = _ln(t, g_ref[...], b_ref[...])
    ao_ref[...] = ao
    lg = _dot(ao, wr_ref[...]) + br_ref[...]
    mx = jnp.max(lg, axis=-1, keepdims=True)
    p = jnp.exp(lg - mx)
    rs_ref[...] = p / jnp.sum(p, axis=-1, keepdims=True)


def _proj_call(ctx, x, Wo, bo, g1, b1, Wr, br):
    return pl.pallas_call(
        _proj_body,
        grid=(S // TB,),
        in_specs=[
            pl.BlockSpec((TB, D), lambda i: (i, 0)),
            pl.BlockSpec((TB, D), lambda i: (i, 0)),
            pl.BlockSpec((D, D), lambda i: (0, 0)),
            pl.BlockSpec((1, D), lambda i: (0, 0)),
            pl.BlockSpec((1, D), lambda i: (0, 0)),
            pl.BlockSpec((1, D), lambda i: (0, 0)),
            pl.BlockSpec((D, E), lambda i: (0, 0)),
            pl.BlockSpec((1, E), lambda i: (0, 0)),
        ],
        out_specs=[
            pl.BlockSpec((TB, D), lambda i: (i, 0)),
            pl.BlockSpec((TB, E), lambda i: (i, 0)),
        ],
        out_shape=[
            jax.ShapeDtypeStruct((S, D), jnp.float32),
            jax.ShapeDtypeStruct((S, E), jnp.float32),
        ],
    )(ctx, x, Wo, bo, g1, b1, Wr, br)


def _rank_body(rs_ref, rl_ref, eid_ref, cnt_ref, acc_ref):
    w = pl.program_id(0)

    @pl.when(w == 0)
    def _():
        acc_ref[...] = jnp.zeros_like(acc_ref)

    rs = rs_ref[...]
    mx = jnp.max(rs, axis=-1, keepdims=True)
    cols = lax.broadcasted_iota(jnp.int32, (TB, E), 1)
    eid = jnp.min(jnp.where(rs >= mx, cols, E), axis=-1)
    onehot = (cols == eid[:, None]).astype(jnp.float32)
    tri = (lax.broadcasted_iota(jnp.int32, (TB, TB), 0)
           > lax.broadcasted_iota(jnp.int32, (TB, TB), 1)).astype(jnp.float32)
    prefix = _dot(tri, onehot)
    base = acc_ref[...]
    rl = jnp.sum(onehot * (prefix + base), axis=-1)
    rl_ref[...] = rl.astype(jnp.int32).reshape(1, 1, TB)
    eid_ref[...] = eid.astype(jnp.int32).reshape(1, 1, TB)
    tot = base + jnp.sum(onehot, axis=0, keepdims=True)
    acc_ref[...] = tot
    cnt_ref[...] = tot.astype(jnp.int32)


def _rank_call(rs):
    return pl.pallas_call(
        _rank_body,
        grid=(S // TB,),
        in_specs=[pl.BlockSpec((TB, E), lambda i: (i, 0))],
        out_specs=[
            pl.BlockSpec((1, 1, TB), lambda i: (i, 0, 0)),
            pl.BlockSpec((1, 1, TB), lambda i: (i, 0, 0)),
            pl.BlockSpec((1, E), lambda i: (0, 0)),
        ],
        out_shape=[
            jax.ShapeDtypeStruct((S // TB, 1, TB), jnp.int32),
            jax.ShapeDtypeStruct((S // TB, 1, TB), jnp.int32),
            jax.ShapeDtypeStruct((1, E), jnp.int32),
        ],
        scratch_shapes=[pltpu.VMEM((1, E), jnp.float32)],
    )(rs)


def _idx_body(cnt_ref, rl_ref, eid_ref, idx_ref, be_ref, bs_ref):
    cv = cnt_ref[...]
    nb = lax.shift_right_logical(cv + (BLK - 1), 7)
    nbf = nb.astype(jnp.float32)
    rows = lax.broadcasted_iota(jnp.int32, (E, E), 0)
    cols = lax.broadcasted_iota(jnp.int32, (E, E), 1)
    upper = (rows <= cols).astype(jnp.float32)
    cumnb = _dot(nbf, upper)
    aoff = ((cumnb - nbf) * float(BLK)).astype(jnp.int32)

    rl = rl_ref[...].reshape(TB, 1)
    eid = eid_ref[...].reshape(TB, 1)
    ecols = lax.broadcasted_iota(jnp.int32, (TB, E), 1)
    onehot = (ecols == eid).astype(jnp.int32)
    idx = rl + jnp.sum(onehot * aoff, axis=-1, keepdims=True)
    idx_ref[...] = idx.reshape(1, 1, TB)

    wv = lax.broadcasted_iota(jnp.int32, (NW, E), 0)
    cb = jnp.broadcast_to(cumnb.astype(jnp.int32), (NW, E))
    acc = jnp.sum((wv >= cb).astype(jnp.int32), axis=-1, keepdims=True)
    wcol = lax.broadcasted_iota(jnp.int32, (NW, 1), 0)
    be_ref[...] = jnp.minimum(acc, E - 1).reshape(1, 1, NW)
    bs_ref[...] = jnp.where(acc < E, wcol, NBLK).reshape(1, 1, NW)


def _idx_call(cnt, rl, eid):
    return pl.pallas_call(
        _idx_body,
        grid=(S // TB,),
        in_specs=[
            pl.BlockSpec((1, E), lambda i: (0, 0)),
            pl.BlockSpec((1, 1, TB), lambda i: (i, 0, 0)),
            pl.BlockSpec((1, 1, TB), lambda i: (i, 0, 0)),
        ],
        out_specs=[
            pl.BlockSpec((1, 1, TB), lambda i: (i, 0, 0)),
            pl.BlockSpec((1, 1, NW), lambda i: (0, 0, 0)),
            pl.BlockSpec((1, 1, NW), lambda i: (0, 0, 0)),
        ],
        out_shape=[
            jax.ShapeDtypeStruct((S // TB, 1, TB), jnp.int32),
            jax.ShapeDtypeStruct((1, 1, NW), jnp.int32),
            jax.ShapeDtypeStruct((1, 1, NW), jnp.int32),
        ],
    )(cnt, rl, eid)


def _disp_body(ao_hbm, idx_hbm, xs_hbm, idx_v, rows_v, sem):
    wid = lax.axis_index("s") * 2 + lax.axis_index("c")
    base = wid * RPW
    pltpu.sync_copy(idx_hbm.at[pl.ds(base, RPW)], idx_v)
    pltpu.sync_copy(ao_hbm.at[pl.ds(base, RPW)], rows_v)
    pltpu.async_copy(rows_v, xs_hbm.at[idx_v], sem).wait()


@functools.lru_cache(maxsize=None)
def _build_dispatch_sc():
    return pl.kernel(
        _disp_body,
        out_type=jax.ShapeDtypeStruct((SP, D), jnp.float32),
        mesh=plsc.VectorSubcoreMesh(core_axis_name="c", subcore_axis_name="s"),
        scratch_types=[
            pltpu.VMEM((RPW,), jnp.int32),
            pltpu.VMEM((RPW, D), jnp.float32),
            pltpu.SemaphoreType.DMA,
        ],
    )


def _dispatch_sc(ao, idx):
    return _build_dispatch_sc()(ao, idx)


def _ffn_body(be_ref, bs_ref, x_ref, wi_ref, bi_ref, wo_ref, bo_ref, g_ref,
              b_ref, o_ref):
    w = pl.program_id(0)

    @pl.when(bs_ref[w] != NBLK)
    def _():
        x = x_ref[...]
        h1 = _dot(x, wi_ref[0]) + bi_ref[0]
        h1 = jax.nn.gelu(h1)
        y = _dot(h1, wo_ref[...]) + bo_ref[...] + x
        o_ref[...] = _ln(y, g_ref[...], b_ref[...])


def _ffn_call(be, bs, xs, Wi, bi, Wout, bout, g2, b2):
    grid_spec = pltpu.PrefetchScalarGridSpec(
        num_scalar_prefetch=2,
        grid=(NBLK,),
        in_specs=[
            pl.BlockSpec((BLK, D), lambda w, be, bs: (bs[w], 0)),
            pl.BlockSpec((1, D, DFF), lambda w, be, bs: (be[w], 0, 0)),
            pl.BlockSpec((1, 1, DFF), lambda w, be, bs: (be[w], 0, 0)),
            pl.BlockSpec((DFF, D), lambda w, be, bs: (0, 0)),
            pl.BlockSpec((1, D), lambda w, be, bs: (0, 0)),
            pl.BlockSpec((1, D), lambda w, be, bs: (0, 0)),
            pl.BlockSpec((1, D), lambda w, be, bs: (0, 0)),
        ],
        out_specs=pl.BlockSpec((BLK, D), lambda w, be, bs: (bs[w], 0)),
    )
    return pl.pallas_call(
        _ffn_body,
        grid_spec=grid_spec,
        out_shape=jax.ShapeDtypeStruct((SP, D), jnp.float32),
    )(be, bs, xs, Wi, bi, Wout, bout, g2, b2)


def _comb_body(os_hbm, rank_hbm, y_hbm, idx_v, rows_v, sem):
    wid = lax.axis_index("s") * 2 + lax.axis_index("c")
    base = wid * RPW
    pltpu.sync_copy(rank_hbm.at[pl.ds(base, RPW)], idx_v)
    pltpu.async_copy(os_hbm.at[idx_v], rows_v, sem).wait()
    pltpu.sync_copy(rows_v, y_hbm.at[pl.ds(base, RPW)])


@functools.lru_cache(maxsize=None)
def _build_combine_sc():
    return pl.kernel(
        _comb_body,
        out_type=jax.ShapeDtypeStruct((S, D), jnp.float32),
        mesh=plsc.VectorSubcoreMesh(core_axis_name="c", subcore_axis_name="s"),
        scratch_types=[
            pltpu.VMEM((RPW,), jnp.int32),
            pltpu.VMEM((RPW, D), jnp.float32),
            pltpu.SemaphoreType.DMA,
        ],
    )


def _combine_sc(os_, rank):
    return _build_combine_sc()(os_, rank)


def kernel(hidden_states, attention_mask, Wq, bq, Wk, bk, Wv, bv, Wo, bo,
           ln1_g, ln1_b, Wr, br, Wi, bi, Wout, bout, ln2_g, ln2_b):
    x = hidden_states.reshape(S, D)
    wqkv = jnp.concatenate([Wq, Wk, Wv], axis=1)
    bqkv = jnp.concatenate([bq, bk, bv]).reshape(1, 3 * D)
    mask = attention_mask.reshape(1, S)

    qkv = _qkv_call(x, wqkv, bqkv)
    ctx = _attn_call(qkv, mask)
    ao, rs = _proj_call(ctx, x, Wo, bo.reshape(1, D), ln1_g.reshape(1, D),
                        ln1_b.reshape(1, D), Wr, br.reshape(1, E))
    rl, eid, cnt = _rank_call(rs)
    idx, be, bs = _idx_call(cnt, rl, eid)
    idx = idx.reshape(S)
    xs = _dispatch_sc(ao, idx)
    os_ = _ffn_call(be.reshape(NW), bs.reshape(NW), xs, Wi,
                    bi.reshape(E, 1, DFF), Wout, bout.reshape(1, D),
                    ln2_g.reshape(1, D), ln2_b.reshape(1, D))
    y = _combine_sc(os_, idx)
    return y.reshape(1, S, D), rs.reshape(1, S, E)


# speedup vs baseline: 2.0267x; 1.0103x over previous
import functools

import jax
import jax.numpy as jnp
from jax import lax
from jax.experimental import pallas as pl
from jax.experimental.pallas import tpu as pltpu
from jax.experimental.pallas import tpu_sc as plsc

S, D, H, DH, DFF, E = 2048, 768, 12, 64, 3072, 8
EPS = 1e-12
TB = 256
BLK = 128
NBLK = 23
SP = (NBLK + 1) * BLK
NW = 32
RPW = S // NW

_PREC = jax.lax.Precision.DEFAULT


def _dot(a, b):
    return jax.lax.dot_general(a, b, (((1,), (0,)), ((), ())),
                               preferred_element_type=jnp.float32,
                               precision=_PREC)


def _ln(t, g, b):
    m = jnp.mean(t, axis=-1, keepdims=True)
    v = jnp.mean((t - m) ** 2, axis=-1, keepdims=True)
    return (t - m) / jnp.sqrt(v + EPS) * g + b


def _qkv_body(x_ref, wq_ref, wk_ref, wv_ref, bq_ref, bk_ref, bv_ref, o_ref):
    x = x_ref[...]
    o_ref[:, 0:D] = _dot(x, wq_ref[...]) + bq_ref[...]
    o_ref[:, D:2 * D] = _dot(x, wk_ref[...]) + bk_ref[...]
    o_ref[:, 2 * D:3 * D] = _dot(x, wv_ref[...]) + bv_ref[...]


def _qkv_call(x, Wq, Wk, Wv, bq, bk, bv):
    wspec = pl.BlockSpec((D, D), lambda i: (0, 0))
    bspec = pl.BlockSpec((1, D), lambda i: (0, 0))
    return pl.pallas_call(
        _qkv_body,
        grid=(S // TB,),
        in_specs=[pl.BlockSpec((TB, D), lambda i: (i, 0)),
                  wspec, wspec, wspec, bspec, bspec, bspec],
        out_specs=pl.BlockSpec((TB, 3 * D), lambda i: (i, 0)),
        out_shape=jax.ShapeDtypeStruct((S, 3 * D), jnp.float32),
    )(x, Wq, Wk, Wv, bq, bk, bv)


ACH = 1024


def _attn_body(q_ref, k_ref, v_ref, m_ref, o_ref):
    q2 = q_ref[...]
    msk = m_ref[...]
    for t in range(2):
        q = q2[:, t * DH:(t + 1) * DH]
        o = m = sm = None
        for c in range(S // ACH):
            kc = k_ref[pl.ds(c * ACH, ACH), t * DH:(t + 1) * DH]
            vc = v_ref[pl.ds(c * ACH, ACH), t * DH:(t + 1) * DH]
            s = jax.lax.dot_general(q, kc, (((1,), (1,)), ((), ())),
                                    preferred_element_type=jnp.float32,
                                    precision=_PREC)
            s = s * (1.0 / 8.0) + msk[:, c * ACH:(c + 1) * ACH]
            cmx = jnp.max(s, axis=-1, keepdims=True)
            if o is None:
                m = cmx
                e = jnp.exp(s - m)
                sm = jnp.sum(e, axis=-1, keepdims=True)
                o = _dot(e, vc)
            else:
                mn = jnp.maximum(m, cmx)
                corr = jnp.exp(m - mn)
                e = jnp.exp(s - mn)
                sm = sm * corr + jnp.sum(e, axis=-1, keepdims=True)
                o = o * corr + _dot(e, vc)
                m = mn
        o_ref[:, t * DH:(t + 1) * DH] = o / sm


def _attn_call(qkv, mask):
    return pl.pallas_call(
        _attn_body,
        grid=(H // 2, S // TB),
        in_specs=[
            pl.BlockSpec((TB, 2 * DH), lambda h, i: (i, h)),
            pl.BlockSpec((S, 2 * DH), lambda h, i: (0, 6 + h)),
            pl.BlockSpec((S, 2 * DH), lambda h, i: (0, 12 + h)),
            pl.BlockSpec((1, S), lambda h, i: (0, 0)),
        ],
        out_specs=pl.BlockSpec((TB, 2 * DH), lambda h, i: (i, h)),
        out_shape=jax.ShapeDtypeStruct((S, D), jnp.float32),
    )(qkv, qkv, qkv, mask)


def _proj_body(ctx_ref, h_ref, wo_ref, bo_ref, g_ref, b_ref, wr_ref, br_ref,
               ao_ref, rs_ref):
    t = _dot(ctx_ref[...], wo_ref[...]) + bo_ref[...] + h_ref[...]
    ao = _ln(t, g_ref[...], b_ref[...])
    ao_ref[...] = ao
    lg = _dot(ao, wr_ref[...]) + br_ref[...]
    mx = jnp.max(lg, axis=-1, keepdims=True)
    p = jnp.exp(lg - mx)
    rs_ref[...] = p / jnp.sum(p, axis=-1, keepdims=True)


def _proj_call(ctx, x, Wo, bo, g1, b1, Wr, br):
    return pl.pallas_call(
        _proj_body,
        grid=(S // TB,),
        in_specs=[
            pl.BlockSpec((TB, D), lambda i: (i, 0)),
            pl.BlockSpec((TB, D), lambda i: (i, 0)),
            pl.BlockSpec((D, D), lambda i: (0, 0)),
            pl.BlockSpec((1, D), lambda i: (0, 0)),
            pl.BlockSpec((1, D), lambda i: (0, 0)),
            pl.BlockSpec((1, D), lambda i: (0, 0)),
            pl.BlockSpec((D, E), lambda i: (0, 0)),
            pl.BlockSpec((1, E), lambda i: (0, 0)),
        ],
        out_specs=[
            pl.BlockSpec((TB, D), lambda i: (i, 0)),
            pl.BlockSpec((TB, E), lambda i: (i, 0)),
        ],
        out_shape=[
            jax.ShapeDtypeStruct((S, D), jnp.float32),
            jax.ShapeDtypeStruct((S, E), jnp.float32),
        ],
    )(ctx, x, Wo, bo, g1, b1, Wr, br)


NB = S // TB


def _route_body(rs_ref, idx_ref, be_ref, bs_ref, acc_ref, rl_s, eid_s):
    w = pl.program_id(0)

    @pl.when(w == 0)
    def _():
        acc_ref[...] = jnp.zeros_like(acc_ref)

    @pl.when(w < NB)
    def _():
        rs = rs_ref[...]
        mx = jnp.max(rs, axis=-1, keepdims=True)
        cols = lax.broadcasted_iota(jnp.int32, (TB, E), 1)
        eid = jnp.min(jnp.where(rs >= mx, cols, E), axis=-1)
        onehot = (cols == eid[:, None]).astype(jnp.float32)
        tri = (lax.broadcasted_iota(jnp.int32, (TB, TB), 0)
               > lax.broadcasted_iota(jnp.int32, (TB, TB), 1)
               ).astype(jnp.float32)
        prefix = _dot(tri, onehot)
        base = acc_ref[...]
        rl = jnp.sum(onehot * (prefix + base), axis=-1).astype(jnp.int32)
        rl_s[pl.ds(w, 1), :] = rl.reshape(1, TB)
        eid_s[pl.ds(w, 1), :] = eid.astype(jnp.int32).reshape(1, TB)
        acc_ref[...] = base + jnp.sum(onehot, axis=0, keepdims=True)

    @pl.when(w >= NB)
    def _():
        b = w - NB
        cv = acc_ref[...].astype(jnp.int32)
        nb = lax.shift_right_logical(cv + (BLK - 1), 7)
        nbf = nb.astype(jnp.float32)
        rows = lax.broadcasted_iota(jnp.int32, (E, E), 0)
        colsE = lax.broadcasted_iota(jnp.int32, (E, E), 1)
        upper = (rows <= colsE).astype(jnp.float32)
        cumnb = _dot(nbf, upper)
        aoff = ((cumnb - nbf) * float(BLK)).astype(jnp.int32)

        rl = rl_s[pl.ds(b, 1), :].reshape(TB, 1)
        eid = eid_s[pl.ds(b, 1), :].reshape(TB, 1)
        ecols = lax.broadcasted_iota(jnp.int32, (TB, E), 1)
        onehot = (ecols == eid).astype(jnp.int32)
        idx = rl + jnp.sum(onehot * aoff, axis=-1, keepdims=True)
        idx_ref[...] = idx.reshape(1, 1, TB)

        @pl.when(w == 2 * NB - 1)
        def _():
            wv = lax.broadcasted_iota(jnp.int32, (NW, E), 0)
            cb = jnp.broadcast_to(cumnb.astype(jnp.int32), (NW, E))
            accb = jnp.sum((wv >= cb).astype(jnp.int32), axis=-1,
                           keepdims=True)
            wcol = lax.broadcasted_iota(jnp.int32, (NW, 1), 0)
            be_ref[...] = jnp.minimum(accb, E - 1).reshape(1, 1, NW)
            bs_ref[...] = jnp.where(accb < E, wcol, NBLK).reshape(1, 1, NW)


def _route_call(rs):
    return pl.pallas_call(
        _route_body,
        grid=(2 * NB,),
        in_specs=[pl.BlockSpec((TB, E), lambda i: (jnp.minimum(i, NB - 1),
                                                   0))],
        out_specs=[
            pl.BlockSpec((1, 1, TB), lambda i: (jnp.maximum(i - NB, 0), 0,
                                                0)),
            pl.BlockSpec((1, 1, NW), lambda i: (0, 0, 0)),
            pl.BlockSpec((1, 1, NW), lambda i: (0, 0, 0)),
        ],
        out_shape=[
            jax.ShapeDtypeStruct((NB, 1, TB), jnp.int32),
            jax.ShapeDtypeStruct((1, 1, NW), jnp.int32),
            jax.ShapeDtypeStruct((1, 1, NW), jnp.int32),
        ],
        scratch_shapes=[
            pltpu.VMEM((1, E), jnp.float32),
            pltpu.VMEM((NB, TB), jnp.int32),
            pltpu.VMEM((NB, TB), jnp.int32),
        ],
    )(rs)


def _disp_body(ao_hbm, idx_hbm, xs_hbm, idx_v, rows_v, sem):
    wid = lax.axis_index("s") * 2 + lax.axis_index("c")
    base = wid * RPW
    pltpu.sync_copy(idx_hbm.at[pl.ds(base, RPW)], idx_v)
    pltpu.sync_copy(ao_hbm.at[pl.ds(base, RPW)], rows_v)
    pltpu.async_copy(rows_v, xs_hbm.at[idx_v], sem).wait()


@functools.lru_cache(maxsize=None)
def _build_dispatch_sc():
    return pl.kernel(
        _disp_body,
        out_type=jax.ShapeDtypeStruct((SP, D), jnp.float32),
        mesh=plsc.VectorSubcoreMesh(core_axis_name="c", subcore_axis_name="s"),
        scratch_types=[
            pltpu.VMEM((RPW,), jnp.int32),
            pltpu.VMEM((RPW, D), jnp.float32),
            pltpu.SemaphoreType.DMA,
        ],
    )


def _dispatch_sc(ao, idx):
    return _build_dispatch_sc()(ao, idx)


def _ffn_body(be_ref, bs_ref, x_ref, wi_ref, bi_ref, wo_ref, bo_ref, g_ref,
              b_ref, o_ref):
    w = pl.program_id(0)

    @pl.when(bs_ref[w] != NBLK)
    def _():
        x = x_ref[...]
        h1 = _dot(x, wi_ref[0]) + bi_ref[0]
        h1 = jax.nn.gelu(h1)
        y = _dot(h1, wo_ref[...]) + bo_ref[...] + x
        o_ref[...] = _ln(y, g_ref[...], b_ref[...])


def _ffn_call(be, bs, xs, Wi, bi, Wout, bout, g2, b2):
    grid_spec = pltpu.PrefetchScalarGridSpec(
        num_scalar_prefetch=2,
        grid=(NBLK,),
        in_specs=[
            pl.BlockSpec((BLK, D), lambda w, be, bs: (bs[w], 0)),
            pl.BlockSpec((1, D, DFF), lambda w, be, bs: (be[w], 0, 0)),
            pl.BlockSpec((1, 1, DFF), lambda w, be, bs: (be[w], 0, 0)),
            pl.BlockSpec((DFF, D), lambda w, be, bs: (0, 0)),
            pl.BlockSpec((1, D), lambda w, be, bs: (0, 0)),
            pl.BlockSpec((1, D), lambda w, be, bs: (0, 0)),
            pl.BlockSpec((1, D), lambda w, be, bs: (0, 0)),
        ],
        out_specs=pl.BlockSpec((BLK, D), lambda w, be, bs: (bs[w], 0)),
    )
    return pl.pallas_call(
        _ffn_body,
        grid_spec=grid_spec,
        out_shape=jax.ShapeDtypeStruct((SP, D), jnp.float32),
    )(be, bs, xs, Wi, bi, Wout, bout, g2, b2)


def _comb_body(os_hbm, rank_hbm, y_hbm, idx_v, rows_v, sem):
    wid = lax.axis_index("s") * 2 + lax.axis_index("c")
    base = wid * RPW
    pltpu.sync_copy(rank_hbm.at[pl.ds(base, RPW)], idx_v)
    pltpu.async_copy(os_hbm.at[idx_v], rows_v, sem).wait()
    pltpu.sync_copy(rows_v, y_hbm.at[pl.ds(base, RPW)])


@functools.lru_cache(maxsize=None)
def _build_combine_sc():
    return pl.kernel(
        _comb_body,
        out_type=jax.ShapeDtypeStruct((S, D), jnp.float32),
        mesh=plsc.VectorSubcoreMesh(core_axis_name="c", subcore_axis_name="s"),
        scratch_types=[
            pltpu.VMEM((RPW,), jnp.int32),
            pltpu.VMEM((RPW, D), jnp.float32),
            pltpu.SemaphoreType.DMA,
        ],
    )


def _combine_sc(os_, rank):
    return _build_combine_sc()(os_, rank)


def kernel(hidden_states, attention_mask, Wq, bq, Wk, bk, Wv, bv, Wo, bo,
           ln1_g, ln1_b, Wr, br, Wi, bi, Wout, bout, ln2_g, ln2_b):
    x = hidden_states.reshape(S, D)
    mask = attention_mask.reshape(1, S)

    qkv = _qkv_call(x, Wq, Wk, Wv, bq.reshape(1, D), bk.reshape(1, D),
                    bv.reshape(1, D))
    ctx = _attn_call(qkv, mask)
    ao, rs = _proj_call(ctx, x, Wo, bo.reshape(1, D), ln1_g.reshape(1, D),
                        ln1_b.reshape(1, D), Wr, br.reshape(1, E))
    idx, be, bs = _route_call(rs)
    idx = idx.reshape(S)
    xs = _dispatch_sc(ao, idx)
    os_ = _ffn_call(be.reshape(NW), bs.reshape(NW), xs, Wi,
                    bi.reshape(E, 1, DFF), Wout, bout.reshape(1, D),
                    ln2_g.reshape(1, D), ln2_b.reshape(1, D))
    y = _combine_sc(os_, idx)
    return y.reshape(1, S, D), rs.reshape(1, S, E)
